# baseline (speedup 1.0000x reference)
import functools

import jax
import jax.numpy as jnp
from jax import lax
from jax.experimental import pallas as pl
from jax.experimental.pallas import tpu as pltpu
from jax.experimental.pallas import tpu_sc as plsc

BATCH = 4096
SEQ = 200
DIM = 32
N = BATCH * SEQ
NC, NS = 2, 16
NW = NC * NS
CHUNK = 128
IDX_BLK = 8
ROWS_PER_W = N // (CHUNK * NW)
SB_PER_W = ROWS_PER_W // IDX_BLK


def _body(idx_hbm, wr_hbm, wi_hbm, out_hbm, idx_v, real_v, imag_v, out_v,
          sem_i, sem_r, sem_m, sem_o):
    wid = lax.axis_index("s") * NC + lax.axis_index("c")
    row0 = wid * ROWS_PER_W

    iota = lax.iota(jnp.int32, 16)
    cols_ra = iota * 2
    cols_rb = iota * 2 + 32
    cols_ia = iota * 2 + 1
    cols_ib = iota * 2 + 33

    def superblock(sb, _):
        base_row = row0 + sb * IDX_BLK
        pltpu.sync_copy(idx_hbm.at[pl.ds(base_row, IDX_BLK)], idx_v)
        for j in range(IDX_BLK):
            cr = pltpu.async_copy(wr_hbm.at[idx_v.at[j]], real_v, sem_r)
            ci = pltpu.async_copy(wi_hbm.at[idx_v.at[j]], imag_v, sem_m)
            cr.wait()
            ci.wait()

            def rows(rg, _):
                for k in range(4):
                    r = rg * 4 + k
                    r64 = r * (2 * DIM)
                    ra = real_v[r, pl.ds(0, 16)]
                    rb = real_v[r, pl.ds(16, 16)]
                    ia = imag_v[r, pl.ds(0, 16)]
                    ib = imag_v[r, pl.ds(16, 16)]
                    plsc.store_scatter(out_v, [cols_ra + r64], ra)
                    plsc.store_scatter(out_v, [cols_rb + r64], rb)
                    plsc.store_scatter(out_v, [cols_ia + r64], ia)
                    plsc.store_scatter(out_v, [cols_ib + r64], ib)
                return 0

            lax.fori_loop(0, CHUNK // 4, rows, 0)
            t0 = (base_row + j) * CHUNK
            pltpu.sync_copy(out_v, out_hbm.at[pl.ds(t0 * 2 * DIM, CHUNK * 2 * DIM)])
        return 0

    lax.fori_loop(0, SB_PER_W, superblock, 0)


@jax.jit
def _dembed(idx2d, w_real, w_imag):
    mesh = plsc.VectorSubcoreMesh(core_axis_name="c", subcore_axis_name="s")
    f = pl.kernel(
        _body,
        out_type=jax.ShapeDtypeStruct((N * 2 * DIM,), jnp.float32),
        mesh=mesh,
        scratch_types=[
            pltpu.VMEM((IDX_BLK, CHUNK), jnp.int32),
            pltpu.VMEM((CHUNK, DIM), jnp.float32),
            pltpu.VMEM((CHUNK, DIM), jnp.float32),
            pltpu.VMEM((CHUNK * 2 * DIM,), jnp.float32),
            pltpu.SemaphoreType.DMA,
            pltpu.SemaphoreType.DMA,
            pltpu.SemaphoreType.DMA,
            pltpu.SemaphoreType.DMA,
        ],
        compiler_params=pltpu.CompilerParams(
            needs_layout_passes=False, use_tc_tiling_on_sc=False),
    )
    return f(idx2d, w_real, w_imag)


def kernel(token_ids, W_real, W_imag):
    ids = token_ids.reshape(N // CHUNK, CHUNK).astype(jnp.int32)
    out = _dembed(ids, W_real, W_imag)
    return out.reshape(BATCH, SEQ, DIM, 2)


# speedup vs baseline: 1.0145x; 1.0145x over previous
import jax
import jax.numpy as jnp
from jax import lax
from jax.experimental import pallas as pl
from jax.experimental.pallas import tpu as pltpu
from jax.experimental.pallas import tpu_sc as plsc

BATCH = 4096
SEQ = 200
DIM = 32
OD = 2 * DIM
N = BATCH * SEQ
NC, NS = 2, 16
NW = NC * NS
CHUNK = 128
IDX_BLK = 8
RING = 8
ROWS_PER_W = N // (CHUNK * NW)
SB_PER_W = ROWS_PER_W // IDX_BLK


def _body(idx_hbm, wr_hbm, wi_hbm, out_hbm, idx_v, real_v, imag_v, out_v,
          sem_i, sem_g, sem_o):
    wid = lax.axis_index("s") * NC + lax.axis_index("c")
    row0 = wid * ROWS_PER_W

    iota = lax.iota(jnp.int32, 16)
    cols = [iota * 2, iota * 2 + 32, iota * 2 + 1, iota * 2 + 33]

    def fetch_idx(sb, slot):
        return pltpu.async_copy(
            idx_hbm.at[pl.ds(row0 + sb * IDX_BLK, IDX_BLK)],
            idx_v.at[slot], sem_i.at[slot])

    def gather(sb_slot, j):
        pltpu.async_copy(wr_hbm.at[idx_v.at[sb_slot, j]],
                         real_v.at[j], sem_g.at[j])
        pltpu.async_copy(wi_hbm.at[idx_v.at[sb_slot, j]],
                         imag_v.at[j], sem_g.at[j])

    def wait_gather(sb_slot, j):
        pltpu.make_async_copy(wr_hbm.at[idx_v.at[sb_slot, j]],
                              real_v.at[j], sem_g.at[j]).wait()
        pltpu.make_async_copy(wi_hbm.at[idx_v.at[sb_slot, j]],
                              imag_v.at[j], sem_g.at[j]).wait()

    def interleave(j, oslot):
        def rows(rg, _):
            for k in range(4):
                r = rg * 4 + k
                r64 = r * OD
                ra = real_v[j, r, pl.ds(0, 16)]
                rb = real_v[j, r, pl.ds(16, 16)]
                ia = imag_v[j, r, pl.ds(0, 16)]
                ib = imag_v[j, r, pl.ds(16, 16)]
                plsc.store_scatter(out_v.at[oslot], [cols[0] + r64], ra)
                plsc.store_scatter(out_v.at[oslot], [cols[1] + r64], rb)
                plsc.store_scatter(out_v.at[oslot], [cols[2] + r64], ia)
                plsc.store_scatter(out_v.at[oslot], [cols[3] + r64], ib)
            return 0
        lax.fori_loop(0, CHUNK // 4, rows, 0, unroll=2)

    def drain_out(t0, oslot):
        pltpu.make_async_copy(out_v.at[oslot],
                              out_hbm.at[pl.ds(t0 * OD, CHUNK * OD)],
                              sem_o.at[oslot]).wait()

    fetch_idx(0, 0).wait()
    for j in range(IDX_BLK):
        gather(0, j)
    fetch_idx(1, 1)

    def superblock(sb, _):
        sb_slot = sb & 1
        for j in range(IDX_BLK):
            wait_gather(sb_slot, j)
            oslot = j & 1
            @pl.when(jnp.logical_or(sb > 0, j >= 2))
            def _():
                jm2 = (j - 2) % IDX_BLK
                sbm = jnp.where(j >= 2, sb, sb - 1)
                drain_out((row0 + sbm * IDX_BLK + jm2) * CHUNK, oslot)

            interleave(j, oslot)

            @pl.when(sb + 1 < SB_PER_W)
            def _():
                if j == 0:
                    pltpu.make_async_copy(
                        idx_hbm.at[pl.ds(row0, IDX_BLK)],
                        idx_v.at[1 - sb_slot], sem_i.at[1 - sb_slot]).wait()
                gather(1 - sb_slot, j)

            @pl.when(sb + 2 < SB_PER_W)
            def _():
                if j == IDX_BLK - 1:
                    fetch_idx(sb + 2, sb_slot)

            t0 = (row0 + sb * IDX_BLK + j) * CHUNK
            pltpu.async_copy(out_v.at[oslot],
                             out_hbm.at[pl.ds(t0 * OD, CHUNK * OD)],
                             sem_o.at[oslot])
        return 0

    lax.fori_loop(0, SB_PER_W, superblock, 0)

    for j in (IDX_BLK - 2, IDX_BLK - 1):
        t0 = (row0 + (SB_PER_W - 1) * IDX_BLK + j) * CHUNK
        drain_out(t0, j & 1)


@jax.jit
def _dembed(idx2d, w_real, w_imag):
    mesh = plsc.VectorSubcoreMesh(core_axis_name="c", subcore_axis_name="s")
    f = pl.kernel(
        _body,
        out_type=jax.ShapeDtypeStruct((N * OD,), jnp.float32),
        mesh=mesh,
        scratch_types=[
            pltpu.VMEM((2, IDX_BLK, CHUNK), jnp.int32),
            pltpu.VMEM((RING, CHUNK, DIM), jnp.float32),
            pltpu.VMEM((RING, CHUNK, DIM), jnp.float32),
            pltpu.VMEM((2, CHUNK * OD), jnp.float32),
            pltpu.SemaphoreType.DMA((2,)),
            pltpu.SemaphoreType.DMA((RING,)),
            pltpu.SemaphoreType.DMA((2,)),
        ],
        compiler_params=pltpu.CompilerParams(
            needs_layout_passes=False, use_tc_tiling_on_sc=False),
    )
    return f(idx2d, w_real, w_imag)


def kernel(token_ids, W_real, W_imag):
    ids = token_ids.reshape(N // CHUNK, CHUNK).astype(jnp.int32)
    out = _dembed(ids, W_real, W_imag)
    return out.reshape(BATCH, SEQ, DIM, 2)


# speedup vs baseline: 10.1934x; 10.0482x over previous
import jax
import jax.numpy as jnp
from jax import lax
from jax.experimental import pallas as pl
from jax.experimental.pallas import tpu as pltpu
from jax.experimental.pallas import tpu_sc as plsc

BATCH = 4096
SEQ = 200
DIM = 32
OD = 2 * DIM
N = BATCH * SEQ
NC, NS = 2, 16
NW = NC * NS
CHUNK = 128
IDX_BLK = 8
RING = 8
SB_PER_W = SEQ // IDX_BLK


def _body(idx_hbm, wr_hbm, wi_hbm, out_hbm, idx_v, real_v, imag_v, stage_v,
          sem_i, sem_g, sem_o):
    wid = lax.axis_index("s") * NC + lax.axis_index("c")
    b0 = wid * CHUNK

    iota = lax.iota(jnp.int32, 16)
    dvec = [iota, iota + 16]

    def fetch_idx(sb, slot):
        return pltpu.async_copy(
            idx_hbm.at[pl.ds(sb * IDX_BLK, IDX_BLK), pl.ds(b0, CHUNK)],
            idx_v.at[slot], sem_i.at[slot])

    def gather(sb_slot, j):
        pltpu.async_copy(wr_hbm.at[idx_v.at[sb_slot, j]],
                         real_v.at[j], sem_g.at[j])
        pltpu.async_copy(wi_hbm.at[idx_v.at[sb_slot, j]],
                         imag_v.at[j], sem_g.at[j])

    def wait_gather(sb_slot, j):
        pltpu.make_async_copy(wr_hbm.at[idx_v.at[sb_slot, j]],
                              real_v.at[j], sem_g.at[j]).wait()
        pltpu.make_async_copy(wi_hbm.at[idx_v.at[sb_slot, j]],
                              imag_v.at[j], sem_g.at[j]).wait()

    def interleave(j, oslot):
        def toks(bg, _):
            for k in range(2):
                bl = bg * 2 + k
                c0 = jnp.full((16,), 0, jnp.int32) + bl
                c1 = c0 + 128
                for h in range(2):
                    ra = real_v[j, bl, pl.ds(16 * h, 16)]
                    ia = imag_v[j, bl, pl.ds(16 * h, 16)]
                    plsc.store_scatter(stage_v.at[oslot], [dvec[h], c0], ra)
                    plsc.store_scatter(stage_v.at[oslot], [dvec[h], c1], ia)
            return 0
        lax.fori_loop(0, CHUNK // 2, toks, 0, unroll=2)

    def out_slab(s):
        return out_hbm.at[s, :, wid, :]

    def drain_out(s, oslot):
        pltpu.make_async_copy(stage_v.at[oslot], out_slab(s),
                              sem_o.at[oslot]).wait()

    fetch_idx(0, 0).wait()
    for j in range(IDX_BLK):
        gather(0, j)
    fetch_idx(1, 1)

    def superblock(sb, _):
        sb_slot = sb & 1
        for j in range(IDX_BLK):
            wait_gather(sb_slot, j)
            oslot = j & 1
            @pl.when(jnp.logical_or(sb > 0, j >= 2))
            def _():
                jm2 = (j - 2) % IDX_BLK
                sbm = jnp.where(j >= 2, sb, sb - 1)
                drain_out(sbm * IDX_BLK + jm2, oslot)

            interleave(j, oslot)

            @pl.when(sb + 1 < SB_PER_W)
            def _():
                if j == 0:
                    pltpu.make_async_copy(
                        idx_hbm.at[pl.ds(0, IDX_BLK), pl.ds(b0, CHUNK)],
                        idx_v.at[1 - sb_slot], sem_i.at[1 - sb_slot]).wait()
                gather(1 - sb_slot, j)

            @pl.when(sb + 2 < SB_PER_W)
            def _():
                if j == IDX_BLK - 1:
                    fetch_idx(sb + 2, sb_slot)

            s = sb * IDX_BLK + j
            pltpu.async_copy(stage_v.at[oslot], out_slab(s), sem_o.at[oslot])
        return 0

    lax.fori_loop(0, SB_PER_W, superblock, 0)

    for j in (IDX_BLK - 2, IDX_BLK - 1):
        drain_out((SB_PER_W - 1) * IDX_BLK + j, j & 1)


@jax.jit
def _dembed(idst, w_real, w_imag):
    mesh = plsc.VectorSubcoreMesh(core_axis_name="c", subcore_axis_name="s")
    f = pl.kernel(
        _body,
        out_type=jax.ShapeDtypeStruct((SEQ, DIM, NW, 2 * CHUNK), jnp.float32),
        mesh=mesh,
        scratch_types=[
            pltpu.VMEM((2, IDX_BLK, CHUNK), jnp.int32),
            pltpu.VMEM((RING, CHUNK, DIM), jnp.float32),
            pltpu.VMEM((RING, CHUNK, DIM), jnp.float32),
            pltpu.VMEM((2, DIM, 2 * CHUNK), jnp.float32),
            pltpu.SemaphoreType.DMA((2,)),
            pltpu.SemaphoreType.DMA((RING,)),
            pltpu.SemaphoreType.DMA((2,)),
        ],
        compiler_params=pltpu.CompilerParams(
            needs_layout_passes=False, use_tc_tiling_on_sc=False),
    )
    return f(idst, w_real, w_imag)


def kernel(token_ids, W_real, W_imag):
    idst = jnp.transpose(token_ids)
    x = _dembed(idst, W_real, W_imag)
    x = x.reshape(SEQ, DIM, NW, 2, CHUNK)
    x = x.transpose(2, 4, 0, 1, 3)
    return x.reshape(BATCH, SEQ, DIM, 2)


# speedup vs baseline: 16.1217x; 1.5816x over previous
import jax
import jax.numpy as jnp
from jax import lax
from jax.experimental import pallas as pl
from jax.experimental.pallas import tpu as pltpu
from jax.experimental.pallas import tpu_sc as plsc

BATCH = 4096
SEQ = 200
DIM = 32
OD = 2 * DIM
N = BATCH * SEQ
NC, NS = 2, 16
NW = NC * NS
CHUNK = 128
IDX_BLK = 8
RING = 8
SB_PER_W = SEQ // IDX_BLK


def _body(idx_hbm, wr_hbm, wi_hbm, out_hbm, idx_v, real_v, imag_v, stage_v,
          sem_i, sem_g, sem_o):
    wid = lax.axis_index("s") * NC + lax.axis_index("c")
    b0 = wid * CHUNK

    iota = lax.iota(jnp.int32, 16)
    dvec = [iota, iota + 16]

    def fetch_idx(sb, slot):
        return pltpu.async_copy(
            idx_hbm.at[pl.ds(sb * IDX_BLK, IDX_BLK), pl.ds(b0, CHUNK)],
            idx_v.at[slot], sem_i.at[slot])

    def gather(sb_slot, j):
        pltpu.async_copy(wr_hbm.at[idx_v.at[sb_slot, j]],
                         real_v.at[j], sem_g.at[j])
        pltpu.async_copy(wi_hbm.at[idx_v.at[sb_slot, j]],
                         imag_v.at[j], sem_g.at[j])

    def wait_gather(sb_slot, j):
        pltpu.make_async_copy(wr_hbm.at[idx_v.at[sb_slot, j]],
                              real_v.at[j], sem_g.at[j]).wait()
        pltpu.make_async_copy(wi_hbm.at[idx_v.at[sb_slot, j]],
                              imag_v.at[j], sem_g.at[j]).wait()

    def interleave(j, oslot):
        def toks(bg, _):
            for k in range(2):
                bl = bg * 2 + k
                c0 = jnp.full((16,), 0, jnp.int32) + bl
                c1 = c0 + 128
                for h in range(2):
                    ra = real_v[j, bl, pl.ds(16 * h, 16)]
                    ia = imag_v[j, bl, pl.ds(16 * h, 16)]
                    plsc.store_scatter(stage_v.at[oslot], [dvec[h], c0], ra)
                    plsc.store_scatter(stage_v.at[oslot], [dvec[h], c1], ia)
            return 0
        lax.fori_loop(0, CHUNK // 2, toks, 0, unroll=2)

    def out_slab(s):
        return out_hbm.at[s, :, wid, :]

    def stage_slab(oslot):
        return stage_v.at[oslot, :, pl.ds(0, 2 * CHUNK)]

    def drain_out(s, oslot):
        pltpu.make_async_copy(stage_slab(oslot), out_slab(s),
                              sem_o.at[oslot]).wait()

    fetch_idx(0, 0).wait()
    for j in range(IDX_BLK):
        gather(0, j)
    fetch_idx(1, 1)

    def superblock(sb, _):
        sb_slot = sb & 1
        for j in range(IDX_BLK):
            wait_gather(sb_slot, j)
            oslot = j & 1
            @pl.when(jnp.logical_or(sb > 0, j >= 2))
            def _():
                jm2 = (j - 2) % IDX_BLK
                sbm = jnp.where(j >= 2, sb, sb - 1)
                drain_out(sbm * IDX_BLK + jm2, oslot)

            interleave(j, oslot)

            @pl.when(sb + 1 < SB_PER_W)
            def _():
                if j == 0:
                    pltpu.make_async_copy(
                        idx_hbm.at[pl.ds(0, IDX_BLK), pl.ds(b0, CHUNK)],
                        idx_v.at[1 - sb_slot], sem_i.at[1 - sb_slot]).wait()
                gather(1 - sb_slot, j)

            @pl.when(sb + 2 < SB_PER_W)
            def _():
                if j == IDX_BLK - 1:
                    fetch_idx(sb + 2, sb_slot)

            s = sb * IDX_BLK + j
            pltpu.async_copy(stage_slab(oslot), out_slab(s), sem_o.at[oslot])
        return 0

    lax.fori_loop(0, SB_PER_W, superblock, 0)

    for j in (IDX_BLK - 2, IDX_BLK - 1):
        drain_out((SB_PER_W - 1) * IDX_BLK + j, j & 1)


@jax.jit
def _dembed(idst, w_real, w_imag):
    mesh = plsc.VectorSubcoreMesh(core_axis_name="c", subcore_axis_name="s")
    f = pl.kernel(
        _body,
        out_type=jax.ShapeDtypeStruct((SEQ, DIM, NW, 2 * CHUNK), jnp.float32),
        mesh=mesh,
        scratch_types=[
            pltpu.VMEM((2, IDX_BLK, CHUNK), jnp.int32),
            pltpu.VMEM((RING, CHUNK, DIM), jnp.float32),
            pltpu.VMEM((RING, CHUNK, DIM), jnp.float32),
            pltpu.VMEM((2, DIM, 2 * CHUNK + 1), jnp.float32),
            pltpu.SemaphoreType.DMA((2,)),
            pltpu.SemaphoreType.DMA((RING,)),
            pltpu.SemaphoreType.DMA((2,)),
        ],
        compiler_params=pltpu.CompilerParams(
            needs_layout_passes=False, use_tc_tiling_on_sc=False),
    )
    return f(idst, w_real, w_imag)


def kernel(token_ids, W_real, W_imag):
    idst = jnp.transpose(token_ids)
    x = _dembed(idst, W_real, W_imag)
    x = x.reshape(SEQ, DIM, NW, 2, CHUNK)
    x = x.transpose(2, 4, 0, 1, 3)
    return x.reshape(BATCH, SEQ, DIM, 2)
